# Initial kernel scaffold; baseline (speedup 1.0000x reference)
import functools

import jax
import jax.numpy as jnp
from jax import lax
from jax.experimental import pallas as pl
from jax.experimental.pallas import tpu as pltpu
from jax.experimental.pallas import tpu_sc as plsc

N = 10000
E = 320000
D = 128

NC = 2
NS = 16
NW = NC * NS
EPW = E // NW
B = 80
WPW = EPW // B
RPT = N // NS
ZR = 25
BR = 400
GRID = N // BR

_mesh = plsc.VectorSubcoreMesh(core_axis_name="c", subcore_axis_name="s")


def _deg_body(dstw_hbm, out_hbm, idx_v, ones_v, zero_v, deg_sh):
    c = lax.axis_index("c")
    s = lax.axis_index("s")
    w = c * NS + s
    pltpu.sync_copy(dstw_hbm.at[w], idx_v)

    def _ones(k, carry):
        ones_v[pl.ds(k * 16, 16)] = jnp.ones((16,), jnp.float32)
        return carry
    lax.fori_loop(0, B // 16, _ones, 0)

    @pl.when(s == 0)
    def _zero():
        def _z(k, carry):
            zero_v[pl.ds(k * 16, 16)] = jnp.zeros((16,), jnp.float32)
            return carry
        lax.fori_loop(0, N // 16, _z, 0)
        pltpu.sync_copy(zero_v, deg_sh)

    plsc.subcore_barrier()

    def _win(j, carry):
        pltpu.sync_copy(ones_v, deg_sh.at[idx_v.at[j]], add=True)
        return carry
    lax.fori_loop(0, WPW, _win, 0)

    plsc.subcore_barrier()

    @pl.when(s == 0)
    def _dump():
        pltpu.sync_copy(deg_sh, out_hbm.at[c])


_deg_call = functools.partial(
    pl.kernel,
    out_type=jax.ShapeDtypeStruct((NC, N), jnp.float32),
    mesh=_mesh,
    scratch_types=[
        pltpu.VMEM((WPW, B), jnp.int32),
        pltpu.VMEM((B,), jnp.float32),
        pltpu.VMEM((N,), jnp.float32),
        pltpu.VMEM_SHARED((N,), jnp.float32),
    ],
)(_deg_body)


def _agg_body(g_hbm, srcw_hbm, dstw_hbm, out_hbm,
              srci_v, dsti_v, rows_v, zbuf_v, acc_sh):
    c = lax.axis_index("c")
    s = lax.axis_index("s")
    w = c * NS + s
    pltpu.sync_copy(srcw_hbm.at[w], srci_v)
    pltpu.sync_copy(dstw_hbm.at[w], dsti_v)

    def _z(i, carry):
        for jj in range(D // 16):
            zbuf_v[i, pl.ds(jj * 16, 16)] = jnp.zeros((16,), jnp.float32)
        return carry
    lax.fori_loop(0, ZR, _z, 0)

    r0 = s * RPT

    def _zc(i, carry):
        pltpu.sync_copy(zbuf_v, acc_sh.at[pl.ds(r0 + i * ZR, ZR)])
        return carry
    lax.fori_loop(0, RPT // ZR, _zc, 0)

    plsc.subcore_barrier()

    def _win(j, carry):
        pltpu.sync_copy(g_hbm.at[srci_v.at[j]], rows_v)
        pltpu.sync_copy(rows_v, acc_sh.at[dsti_v.at[j]], add=True)
        return carry
    lax.fori_loop(0, WPW, _win, 0)

    plsc.subcore_barrier()
    pltpu.sync_copy(acc_sh.at[pl.ds(r0, RPT)], out_hbm.at[c, pl.ds(r0, RPT)])


_agg_call = functools.partial(
    pl.kernel,
    out_type=jax.ShapeDtypeStruct((NC, N, D), jnp.float32),
    mesh=_mesh,
    scratch_types=[
        pltpu.VMEM((WPW, B), jnp.int32),
        pltpu.VMEM((WPW, B), jnp.int32),
        pltpu.VMEM((B, D), jnp.float32),
        pltpu.VMEM((ZR, D), jnp.float32),
        pltpu.VMEM_SHARED((N, D), jnp.float32),
    ],
)(_agg_body)


def _dinv_block(degpT_ref):
    deg = 1.0 + jnp.sum(degpT_ref[...], axis=1, keepdims=True)
    return lax.rsqrt(deg)


def _mm1_body(degpT_ref, x_ref, w_ref, o_ref):
    dinv = _dinv_block(degpT_ref)
    h = jnp.dot(x_ref[...], w_ref[...], preferred_element_type=jnp.float32)
    o_ref[...] = h * dinv


def _l2_body(degpT_ref, p_ref, g_ref, b_ref, w_ref, o_ref):
    dinv = _dinv_block(degpT_ref)
    ssum = p_ref[0] + p_ref[1] + g_ref[...]
    h1 = jnp.maximum(ssum * dinv + b_ref[...], 0.0)
    o_ref[...] = jnp.dot(h1, w_ref[...],
                         preferred_element_type=jnp.float32) * dinv


def _fin_body(degpT_ref, p_ref, g_ref, b_ref, o_ref):
    dinv = _dinv_block(degpT_ref)
    ssum = p_ref[0] + p_ref[1] + g_ref[...]
    o_ref[...] = ssum * dinv + b_ref[...]


_degpT_spec = pl.BlockSpec((BR, NC), lambda i: (i, 0))
_row_spec = pl.BlockSpec((BR, D), lambda i: (i, 0))
_p_spec = pl.BlockSpec((NC, BR, D), lambda i: (0, i, 0))
_w_spec = pl.BlockSpec((D, D), lambda i: (0, 0))
_b_spec = pl.BlockSpec((1, D), lambda i: (0, 0))
_out_struct = jax.ShapeDtypeStruct((N, D), jnp.float32)

_mm1 = pl.pallas_call(
    _mm1_body,
    grid=(GRID,),
    in_specs=[_degpT_spec, _row_spec, _w_spec],
    out_specs=_row_spec,
    out_shape=_out_struct,
)

_l2 = pl.pallas_call(
    _l2_body,
    grid=(GRID,),
    in_specs=[_degpT_spec, _p_spec, _row_spec, _b_spec, _w_spec],
    out_specs=_row_spec,
    out_shape=_out_struct,
)

_fin = pl.pallas_call(
    _fin_body,
    grid=(GRID,),
    in_specs=[_degpT_spec, _p_spec, _row_spec, _b_spec],
    out_specs=_row_spec,
    out_shape=_out_struct,
)


def kernel(x, edge_index, W1, b1, W2, b2):
    src = edge_index[0].reshape(NW, WPW, B)
    dst = edge_index[1].reshape(NW, WPW, B)
    degp = _deg_call(dst)
    degpT = degp.T
    g1 = _mm1(degpT, x, W1)
    p1 = _agg_call(g1, src, dst)
    g2 = _l2(degpT, p1, g1, b1.reshape(1, D), W2)
    p2 = _agg_call(g2, src, dst)
    return _fin(degpT, p2, g2, b2.reshape(1, D))


# speedup vs baseline: 16.9609x; 16.9609x over previous
import functools

import jax
import jax.numpy as jnp
from jax import lax
from jax.experimental import pallas as pl
from jax.experimental.pallas import tpu as pltpu
from jax.experimental.pallas import tpu_sc as plsc

N = 10000
E = 320000
D = 128

NC = 2
NS = 16
NW = NC * NS
EPW = E // NW
B = 80
WPW = EPW // B
NPAD = 10240
RPT = NPAD // NS
ZR = 40
BR = 400
GRID = N // BR

_mesh = plsc.VectorSubcoreMesh(core_axis_name="c", subcore_axis_name="s")


DW = 16


def _deg_body(dstw_hbm, out_hbm, idx_v, ones_v, zbuf_v, deg_sh):
    c = lax.axis_index("c")
    s = lax.axis_index("s")
    w = c * NS + s
    pltpu.sync_copy(dstw_hbm.at[w], idx_v)

    def _ones(k, carry):
        ones_v[k, :] = jnp.ones((DW,), jnp.float32)
        return carry
    lax.fori_loop(0, B, _ones, 0)

    def _z(i, carry):
        zbuf_v[i, :] = jnp.zeros((DW,), jnp.float32)
        return carry
    lax.fori_loop(0, ZR, _z, 0)

    r0 = s * RPT

    def _zc(i, carry):
        pltpu.sync_copy(zbuf_v, deg_sh.at[pl.ds(r0 + i * ZR, ZR)])
        return carry
    lax.fori_loop(0, RPT // ZR, _zc, 0)

    plsc.subcore_barrier()

    def _win(j, carry):
        pltpu.sync_copy(ones_v, deg_sh.at[idx_v.at[j]], add=True)
        return carry
    lax.fori_loop(0, WPW, _win, 0)

    plsc.subcore_barrier()

    @pl.when(s < NS - 1)
    def _dump_full():
        pltpu.sync_copy(deg_sh.at[pl.ds(r0, RPT)],
                        out_hbm.at[c, pl.ds(r0, RPT)])

    @pl.when(s == NS - 1)
    def _dump_tail():
        pltpu.sync_copy(deg_sh.at[pl.ds(r0, N - (NS - 1) * RPT)],
                        out_hbm.at[c, pl.ds(r0, N - (NS - 1) * RPT)])


_deg_call = functools.partial(
    pl.kernel,
    out_type=jax.ShapeDtypeStruct((NC, N, DW), jnp.float32),
    mesh=_mesh,
    scratch_types=[
        pltpu.VMEM((WPW, B), jnp.int32),
        pltpu.VMEM((B, DW), jnp.float32),
        pltpu.VMEM((ZR, DW), jnp.float32),
        pltpu.VMEM_SHARED((NPAD, DW), jnp.float32),
    ],
)(_deg_body)


def _agg_body(g_hbm, srcw_hbm, dstw_hbm, out_hbm,
              srci_v, dsti_v, rows_v, zbuf_v, acc_sh):
    c = lax.axis_index("c")
    s = lax.axis_index("s")
    w = c * NS + s
    pltpu.sync_copy(srcw_hbm.at[w], srci_v)
    pltpu.sync_copy(dstw_hbm.at[w], dsti_v)

    def _z(i, carry):
        for jj in range(D // 16):
            zbuf_v[i, pl.ds(jj * 16, 16)] = jnp.zeros((16,), jnp.float32)
        return carry
    lax.fori_loop(0, ZR, _z, 0)

    r0 = s * RPT

    def _zc(i, carry):
        pltpu.sync_copy(zbuf_v, acc_sh.at[pl.ds(r0 + i * ZR, ZR)])
        return carry
    lax.fori_loop(0, RPT // ZR, _zc, 0)

    plsc.subcore_barrier()

    def _win(j, carry):
        pltpu.sync_copy(g_hbm.at[srci_v.at[j]], rows_v)
        pltpu.sync_copy(rows_v, acc_sh.at[dsti_v.at[j]], add=True)
        return carry
    lax.fori_loop(0, WPW, _win, 0)

    plsc.subcore_barrier()

    @pl.when(s < NS - 1)
    def _dump_full():
        pltpu.sync_copy(acc_sh.at[pl.ds(r0, RPT)],
                        out_hbm.at[c, pl.ds(r0, RPT)])

    @pl.when(s == NS - 1)
    def _dump_tail():
        pltpu.sync_copy(acc_sh.at[pl.ds(r0, N - (NS - 1) * RPT)],
                        out_hbm.at[c, pl.ds(r0, N - (NS - 1) * RPT)])


_agg_call = functools.partial(
    pl.kernel,
    out_type=jax.ShapeDtypeStruct((NC, N, D), jnp.float32),
    mesh=_mesh,
    scratch_types=[
        pltpu.VMEM((WPW, B), jnp.int32),
        pltpu.VMEM((WPW, B), jnp.int32),
        pltpu.VMEM((B, D), jnp.float32),
        pltpu.VMEM((ZR, D), jnp.float32),
        pltpu.VMEM_SHARED((NPAD, D), jnp.float32),
    ],
)(_agg_body)


def _dinv_block(degpT_ref):
    deg = 1.0 + jnp.sum(degpT_ref[...], axis=1, keepdims=True)
    return lax.rsqrt(deg)


def _mm1_body(degpT_ref, x_ref, w_ref, o_ref):
    dinv = _dinv_block(degpT_ref)
    h = jnp.dot(x_ref[...], w_ref[...], preferred_element_type=jnp.float32)
    o_ref[...] = h * dinv


def _l2_body(degpT_ref, p_ref, g_ref, b_ref, w_ref, o_ref):
    dinv = _dinv_block(degpT_ref)
    ssum = p_ref[0] + p_ref[1] + g_ref[...]
    h1 = jnp.maximum(ssum * dinv + b_ref[...], 0.0)
    o_ref[...] = jnp.dot(h1, w_ref[...],
                         preferred_element_type=jnp.float32) * dinv


def _fin_body(degpT_ref, p_ref, g_ref, b_ref, o_ref):
    dinv = _dinv_block(degpT_ref)
    ssum = p_ref[0] + p_ref[1] + g_ref[...]
    o_ref[...] = ssum * dinv + b_ref[...]


_degpT_spec = pl.BlockSpec((BR, NC), lambda i: (i, 0))
_row_spec = pl.BlockSpec((BR, D), lambda i: (i, 0))
_p_spec = pl.BlockSpec((NC, BR, D), lambda i: (0, i, 0))
_w_spec = pl.BlockSpec((D, D), lambda i: (0, 0))
_b_spec = pl.BlockSpec((1, D), lambda i: (0, 0))
_out_struct = jax.ShapeDtypeStruct((N, D), jnp.float32)

_mm1 = pl.pallas_call(
    _mm1_body,
    grid=(GRID,),
    in_specs=[_degpT_spec, _row_spec, _w_spec],
    out_specs=_row_spec,
    out_shape=_out_struct,
)

_l2 = pl.pallas_call(
    _l2_body,
    grid=(GRID,),
    in_specs=[_degpT_spec, _p_spec, _row_spec, _b_spec, _w_spec],
    out_specs=_row_spec,
    out_shape=_out_struct,
)

_fin = pl.pallas_call(
    _fin_body,
    grid=(GRID,),
    in_specs=[_degpT_spec, _p_spec, _row_spec, _b_spec],
    out_specs=_row_spec,
    out_shape=_out_struct,
)


def kernel(x, edge_index, W1, b1, W2, b2):
    src = edge_index[0].reshape(NW, WPW, B)
    dst = edge_index[1].reshape(NW, WPW, B)
    degp = _deg_call(dst)
    degpT = degp[:, :, 0].T
    g1 = _mm1(degpT, x, W1)
    p1 = _agg_call(g1, src, dst)
    g2 = _l2(degpT, p1, g1, b1.reshape(1, D), W2)
    p2 = _agg_call(g2, src, dst)
    return _fin(degpT, p2, g2, b2.reshape(1, D))


# speedup vs baseline: 22.7251x; 1.3399x over previous
import functools

import jax
import jax.numpy as jnp
from jax import lax
from jax.experimental import pallas as pl
from jax.experimental.pallas import tpu as pltpu
from jax.experimental.pallas import tpu_sc as plsc

N = 10000
E = 320000
D = 128

NC = 2
NS = 16
NW = NC * NS
EPW = E // NW
B = 125
WPW = EPW // B
CH = 8
NCHUNK = WPW // CH
NPAD = 10240
RPT = NPAD // NS
BR = 400
GRID = N // BR

_mesh = plsc.VectorSubcoreMesh(core_axis_name="c", subcore_axis_name="s")


DW = 16
IQ = 8


def _deg_body(dstw_hbm, out_hbm, idx_v, ones_v, zbuf_v, deg_sh, dsem):
    c = lax.axis_index("c")
    s = lax.axis_index("s")
    w = c * NS + s
    pltpu.sync_copy(dstw_hbm.at[w], idx_v)

    def _ones(k, carry):
        ones_v[k, :] = jnp.ones((DW,), jnp.float32)
        return carry
    lax.fori_loop(0, B, _ones, 0)

    def _z(i, carry):
        zbuf_v[i, :] = jnp.zeros((DW,), jnp.float32)
        return carry
    lax.fori_loop(0, 40, _z, 0)

    r0 = s * RPT

    def _zc(i, carry):
        pltpu.sync_copy(zbuf_v, deg_sh.at[pl.ds(r0 + i * 40, 40)])
        return carry
    lax.fori_loop(0, RPT // 40, _zc, 0)

    plsc.subcore_barrier()

    def _win(j, carry):
        pltpu.sync_copy(ones_v, deg_sh.at[idx_v.at[j]], add=True)
        return carry
    lax.fori_loop(0, WPW, _win, 0)

    plsc.subcore_barrier()

    @pl.when(s < NS - 1)
    def _dump_full():
        pltpu.sync_copy(deg_sh.at[pl.ds(r0, RPT)],
                        out_hbm.at[c, pl.ds(r0, RPT)])

    @pl.when(s == NS - 1)
    def _dump_tail():
        pltpu.sync_copy(deg_sh.at[pl.ds(r0, N - (NS - 1) * RPT)],
                        out_hbm.at[c, pl.ds(r0, N - (NS - 1) * RPT)])


_deg_call = functools.partial(
    pl.kernel,
    out_type=jax.ShapeDtypeStruct((NC, N, DW), jnp.float32),
    mesh=_mesh,
    scratch_types=[
        pltpu.VMEM((WPW, B), jnp.int32),
        pltpu.VMEM((B, DW), jnp.float32),
        pltpu.VMEM((40, DW), jnp.float32),
        pltpu.VMEM_SHARED((NPAD, DW), jnp.float32),
        pltpu.SemaphoreType.DMA,
    ],
)(_deg_body)


def _agg_body(g_hbm, srcw_hbm, dstw_hbm, out_hbm,
              srci_v, dstc_v, rows_v, acc_sh, gsem0, gsem1, csem0, csem1):
    c = lax.axis_index("c")
    s = lax.axis_index("s")
    w = c * NS + s
    pltpu.sync_copy(srcw_hbm.at[w], srci_v)
    pltpu.sync_copy(dstw_hbm.at[w, 0], dstc_v.at[0])

    def _z(i, carry):
        for jj in range(D // 16):
            rows_v[0, i, pl.ds(jj * 16, 16)] = jnp.zeros((16,), jnp.float32)
        return carry
    lax.fori_loop(0, 64, _z, 0)

    r0 = s * RPT

    def _zc(i, carry):
        pltpu.sync_copy(rows_v.at[0, pl.ds(0, 64)],
                        acc_sh.at[pl.ds(r0 + i * 64, 64)])
        return carry
    lax.fori_loop(0, RPT // 64, _zc, 0)

    plsc.subcore_barrier()

    gsems = (gsem0, gsem1)
    csems = (csem0, csem1)

    def _start(j, k):
        pltpu.async_copy(g_hbm.at[srci_v.at[j]], rows_v.at[k], gsems[k])

    def _gwait(k):
        pltpu.make_async_copy(g_hbm.at[srci_v.at[0]], rows_v.at[k],
                              gsems[k]).wait()

    def _scat(slot, i, k):
        pltpu.sync_copy(rows_v.at[k], acc_sh.at[dstc_v.at[slot, i]],
                        add=True)

    def _cstart(ch, slot):
        pltpu.async_copy(dstw_hbm.at[w, ch], dstc_v.at[slot], csems[slot])

    def _cwait(slot):
        pltpu.make_async_copy(dstw_hbm.at[w, 0], dstc_v.at[slot],
                              csems[slot]).wait()

    def _overlap(jg, guard, kg, slot, i_scat, k_scat):
        def scoped(sem):
            @pl.when(guard)
            def _():
                pltpu.async_copy(g_hbm.at[srci_v.at[jg]],
                                 rows_v.at[kg], sem)
            _scat(slot, i_scat, k_scat)

            @pl.when(guard)
            def _():
                pltpu.make_async_copy(g_hbm.at[srci_v.at[jg]],
                                      rows_v.at[kg], sem).wait()
        pl.run_scoped(scoped, pltpu.SemaphoreType.DMA)

    def _chunk(ch, slot):
        def _pair(i, carry):
            j0 = ch * CH + 2 * i
            _overlap(j0 + 1, j0 + 1 < WPW, 1, slot, 2 * i, 0)
            _overlap(j0 + 2, j0 + 2 < WPW, 0, slot, 2 * i + 1, 1)
            return carry
        lax.fori_loop(0, CH // 2, _pair, 0)

    pltpu.sync_copy(g_hbm.at[srci_v.at[0]], rows_v.at[0])

    def _chunk2(t, carry):
        _chunk(2 * t, 0)
        _cstart(2 * t + 1, 1)
        _cwait(1)
        _chunk(2 * t + 1, 1)

        @pl.when(t + 1 < NCHUNK // 2)
        def _():
            _cstart(2 * t + 2, 0)
            _cwait(0)
        return carry
    lax.fori_loop(0, NCHUNK // 2, _chunk2, 0)

    plsc.subcore_barrier()

    @pl.when(s < NS - 1)
    def _dump_full():
        pltpu.sync_copy(acc_sh.at[pl.ds(r0, RPT)],
                        out_hbm.at[c, pl.ds(r0, RPT)])

    @pl.when(s == NS - 1)
    def _dump_tail():
        pltpu.sync_copy(acc_sh.at[pl.ds(r0, N - (NS - 1) * RPT)],
                        out_hbm.at[c, pl.ds(r0, N - (NS - 1) * RPT)])


_agg_call = functools.partial(
    pl.kernel,
    out_type=jax.ShapeDtypeStruct((NC, N, D), jnp.float32),
    mesh=_mesh,
    scratch_types=[
        pltpu.VMEM((WPW, B), jnp.int32),
        pltpu.VMEM((2, CH, B), jnp.int32),
        pltpu.VMEM((2, B, D), jnp.float32),
        pltpu.VMEM_SHARED((NPAD, D), jnp.float32),
        pltpu.SemaphoreType.DMA,
        pltpu.SemaphoreType.DMA,
        pltpu.SemaphoreType.DMA,
        pltpu.SemaphoreType.DMA,
    ],
)(_agg_body)


def _dinv_block(degpT_ref):
    deg = 1.0 + jnp.sum(degpT_ref[...], axis=1, keepdims=True)
    return lax.rsqrt(deg)


def _mm1_body(degpT_ref, x_ref, w_ref, o_ref):
    dinv = _dinv_block(degpT_ref)
    h = jnp.dot(x_ref[...], w_ref[...], preferred_element_type=jnp.float32)
    o_ref[...] = h * dinv


def _l2_body(degpT_ref, p_ref, g_ref, b_ref, w_ref, o_ref):
    dinv = _dinv_block(degpT_ref)
    ssum = p_ref[0] + p_ref[1] + g_ref[...]
    h1 = jnp.maximum(ssum * dinv + b_ref[...], 0.0)
    o_ref[...] = jnp.dot(h1, w_ref[...],
                         preferred_element_type=jnp.float32) * dinv


def _fin_body(degpT_ref, p_ref, g_ref, b_ref, o_ref):
    dinv = _dinv_block(degpT_ref)
    ssum = p_ref[0] + p_ref[1] + g_ref[...]
    o_ref[...] = ssum * dinv + b_ref[...]


_degpT_spec = pl.BlockSpec((BR, NC), lambda i: (i, 0))
_row_spec = pl.BlockSpec((BR, D), lambda i: (i, 0))
_p_spec = pl.BlockSpec((NC, BR, D), lambda i: (0, i, 0))
_w_spec = pl.BlockSpec((D, D), lambda i: (0, 0))
_b_spec = pl.BlockSpec((1, D), lambda i: (0, 0))
_out_struct = jax.ShapeDtypeStruct((N, D), jnp.float32)

_mm1 = pl.pallas_call(
    _mm1_body,
    grid=(GRID,),
    in_specs=[_degpT_spec, _row_spec, _w_spec],
    out_specs=_row_spec,
    out_shape=_out_struct,
)

_l2 = pl.pallas_call(
    _l2_body,
    grid=(GRID,),
    in_specs=[_degpT_spec, _p_spec, _row_spec, _b_spec, _w_spec],
    out_specs=_row_spec,
    out_shape=_out_struct,
)

_fin = pl.pallas_call(
    _fin_body,
    grid=(GRID,),
    in_specs=[_degpT_spec, _p_spec, _row_spec, _b_spec],
    out_specs=_row_spec,
    out_shape=_out_struct,
)


def kernel(x, edge_index, W1, b1, W2, b2):
    src = edge_index[0].reshape(NW, WPW, B)
    dst = edge_index[1].reshape(NW, NCHUNK, CH, B)
    dstw = edge_index[1].reshape(NW, WPW, B)
    degp = _deg_call(dstw)
    degpT = degp[:, :, 0].T
    g1 = _mm1(degpT, x, W1)
    p1 = _agg_call(g1, src, dst)
    g2 = _l2(degpT, p1, g1, b1.reshape(1, D), W2)
    p2 = _agg_call(g2, src, dst)
    return _fin(degpT, p2, g2, b2.reshape(1, D))


# speedup vs baseline: 26.3132x; 1.1579x over previous
import functools

import jax
import jax.numpy as jnp
from jax import lax
from jax.experimental import pallas as pl
from jax.experimental.pallas import tpu as pltpu
from jax.experimental.pallas import tpu_sc as plsc

N = 10000
E = 320000
D = 128

NC = 2
NS = 16
NW = NC * NS
EPW = E // NW
B = 125
WPW = EPW // B
CH = 8
NCHUNK = WPW // CH
NPAD = 10240
RPT = NPAD // NS
BR = 400
GRID = N // BR

_mesh = plsc.VectorSubcoreMesh(core_axis_name="c", subcore_axis_name="s")


DW = 16
IQ = 8


def _deg_body(dstw_hbm, out_hbm, idx_v, ones_v, zbuf_v, deg_sh):
    c = lax.axis_index("c")
    s = lax.axis_index("s")
    w = c * NS + s
    pltpu.sync_copy(dstw_hbm.at[w], idx_v)

    def _ones(k, carry):
        ones_v[k, :] = jnp.ones((DW,), jnp.float32)
        return carry
    lax.fori_loop(0, B, _ones, 0)

    def _z(i, carry):
        zbuf_v[i, :] = jnp.zeros((DW,), jnp.float32)
        return carry
    lax.fori_loop(0, 40, _z, 0)

    r0 = s * RPT

    def _zc(i, carry):
        pltpu.sync_copy(zbuf_v, deg_sh.at[pl.ds(r0 + i * 40, 40)])
        return carry
    lax.fori_loop(0, RPT // 40, _zc, 0)

    plsc.subcore_barrier()

    def _win(ch, carry):
        def scoped(*sems):
            for k in range(CH):
                pltpu.async_copy(ones_v, deg_sh.at[idx_v.at[ch, k]],
                                 sems[k], add=True)
            for k in range(CH):
                pltpu.make_async_copy(ones_v, deg_sh.at[idx_v.at[ch, k]],
                                      sems[k]).wait()
        pl.run_scoped(scoped, *([pltpu.SemaphoreType.DMA] * CH))
        return carry
    lax.fori_loop(0, NCHUNK, _win, 0)

    plsc.subcore_barrier()

    @pl.when(s < NS - 1)
    def _dump_full():
        pltpu.sync_copy(deg_sh.at[pl.ds(r0, RPT)],
                        out_hbm.at[c, pl.ds(r0, RPT)])

    @pl.when(s == NS - 1)
    def _dump_tail():
        pltpu.sync_copy(deg_sh.at[pl.ds(r0, N - (NS - 1) * RPT)],
                        out_hbm.at[c, pl.ds(r0, N - (NS - 1) * RPT)])


_deg_call = functools.partial(
    pl.kernel,
    out_type=jax.ShapeDtypeStruct((NC, N, DW), jnp.float32),
    mesh=_mesh,
    scratch_types=[
        pltpu.VMEM((NCHUNK, CH, B), jnp.int32),
        pltpu.VMEM((B, DW), jnp.float32),
        pltpu.VMEM((40, DW), jnp.float32),
        pltpu.VMEM_SHARED((NPAD, DW), jnp.float32),
    ],
)(_deg_body)


def _agg_body(g_hbm, srcw_hbm, dstw_hbm, out_hbm,
              srci_v, dstc_v, rows_v, acc_sh, gsem0, gsem1, csem0, csem1):
    c = lax.axis_index("c")
    s = lax.axis_index("s")
    w = c * NS + s
    pltpu.sync_copy(srcw_hbm.at[w], srci_v)
    pltpu.sync_copy(dstw_hbm.at[w, 0], dstc_v.at[0])

    def _z(i, carry):
        for jj in range(D // 16):
            rows_v[0, i, pl.ds(jj * 16, 16)] = jnp.zeros((16,), jnp.float32)
        return carry
    lax.fori_loop(0, 64, _z, 0)

    r0 = s * RPT

    def _zc(i, carry):
        pltpu.sync_copy(rows_v.at[0, pl.ds(0, 64)],
                        acc_sh.at[pl.ds(r0 + i * 64, 64)])
        return carry
    lax.fori_loop(0, RPT // 64, _zc, 0)

    plsc.subcore_barrier()

    gsems = (gsem0, gsem1)
    csems = (csem0, csem1)

    def _start(j, k):
        pltpu.async_copy(g_hbm.at[srci_v.at[j]], rows_v.at[k], gsems[k])

    def _gwait(k):
        pltpu.make_async_copy(g_hbm.at[srci_v.at[0]], rows_v.at[k],
                              gsems[k]).wait()

    def _scat(slot, i, k):
        pltpu.sync_copy(rows_v.at[k], acc_sh.at[dstc_v.at[slot, i]],
                        add=True)

    def _cstart(ch, slot):
        pltpu.async_copy(dstw_hbm.at[w, ch], dstc_v.at[slot], csems[slot])

    def _cwait(slot):
        pltpu.make_async_copy(dstw_hbm.at[w, 0], dstc_v.at[slot],
                              csems[slot]).wait()

    def _overlap(jg, guard, kg, slot, i_scat, k_scat):
        def scoped(sem):
            @pl.when(guard)
            def _():
                pltpu.async_copy(g_hbm.at[srci_v.at[jg]],
                                 rows_v.at[kg], sem)
            _scat(slot, i_scat, k_scat)

            @pl.when(guard)
            def _():
                pltpu.make_async_copy(g_hbm.at[srci_v.at[jg]],
                                      rows_v.at[kg], sem).wait()
        pl.run_scoped(scoped, pltpu.SemaphoreType.DMA)

    def _chunk(ch, slot):
        def _pair(i, carry):
            j0 = ch * CH + 2 * i
            _overlap(j0 + 1, j0 + 1 < WPW, 1, slot, 2 * i, 0)
            _overlap(j0 + 2, j0 + 2 < WPW, 0, slot, 2 * i + 1, 1)
            return carry
        lax.fori_loop(0, CH // 2, _pair, 0)

    pltpu.sync_copy(g_hbm.at[srci_v.at[0]], rows_v.at[0])

    def _chunk2(t, carry):
        _chunk(2 * t, 0)
        _cstart(2 * t + 1, 1)
        _cwait(1)
        _chunk(2 * t + 1, 1)

        @pl.when(t + 1 < NCHUNK // 2)
        def _():
            _cstart(2 * t + 2, 0)
            _cwait(0)
        return carry
    lax.fori_loop(0, NCHUNK // 2, _chunk2, 0)

    plsc.subcore_barrier()

    @pl.when(s < NS - 1)
    def _dump_full():
        pltpu.sync_copy(acc_sh.at[pl.ds(r0, RPT)],
                        out_hbm.at[c, pl.ds(r0, RPT)])

    @pl.when(s == NS - 1)
    def _dump_tail():
        pltpu.sync_copy(acc_sh.at[pl.ds(r0, N - (NS - 1) * RPT)],
                        out_hbm.at[c, pl.ds(r0, N - (NS - 1) * RPT)])


_agg_call = functools.partial(
    pl.kernel,
    out_type=jax.ShapeDtypeStruct((NC, N, D), jnp.float32),
    mesh=_mesh,
    scratch_types=[
        pltpu.VMEM((WPW, B), jnp.int32),
        pltpu.VMEM((2, CH, B), jnp.int32),
        pltpu.VMEM((2, B, D), jnp.float32),
        pltpu.VMEM_SHARED((NPAD, D), jnp.float32),
        pltpu.SemaphoreType.DMA,
        pltpu.SemaphoreType.DMA,
        pltpu.SemaphoreType.DMA,
        pltpu.SemaphoreType.DMA,
    ],
)(_agg_body)


def _dinv_block(degp_ref):
    deg = 1.0 + degp_ref[0, :, 0:1] + degp_ref[1, :, 0:1]
    return lax.rsqrt(deg)


def _mm1_body(degp_ref, x_ref, w_ref, o_ref):
    dinv = _dinv_block(degp_ref)
    h = jnp.dot(x_ref[...], w_ref[...], preferred_element_type=jnp.float32)
    o_ref[...] = h * dinv


def _l2_body(degp_ref, p_ref, g_ref, b_ref, w_ref, o_ref):
    dinv = _dinv_block(degp_ref)
    ssum = p_ref[0] + p_ref[1] + g_ref[...]
    h1 = jnp.maximum(ssum * dinv + b_ref[...], 0.0)
    o_ref[...] = jnp.dot(h1, w_ref[...],
                         preferred_element_type=jnp.float32) * dinv


def _fin_body(degp_ref, p_ref, g_ref, b_ref, o_ref):
    dinv = _dinv_block(degp_ref)
    ssum = p_ref[0] + p_ref[1] + g_ref[...]
    o_ref[...] = ssum * dinv + b_ref[...]


_degp_spec = pl.BlockSpec((NC, BR, DW), lambda i: (0, i, 0))
_row_spec = pl.BlockSpec((BR, D), lambda i: (i, 0))
_p_spec = pl.BlockSpec((NC, BR, D), lambda i: (0, i, 0))
_w_spec = pl.BlockSpec((D, D), lambda i: (0, 0))
_b_spec = pl.BlockSpec((1, D), lambda i: (0, 0))
_out_struct = jax.ShapeDtypeStruct((N, D), jnp.float32)

_mm1 = pl.pallas_call(
    _mm1_body,
    grid=(GRID,),
    in_specs=[_degp_spec, _row_spec, _w_spec],
    out_specs=_row_spec,
    out_shape=_out_struct,
)

_l2 = pl.pallas_call(
    _l2_body,
    grid=(GRID,),
    in_specs=[_degp_spec, _p_spec, _row_spec, _b_spec, _w_spec],
    out_specs=_row_spec,
    out_shape=_out_struct,
)

_fin = pl.pallas_call(
    _fin_body,
    grid=(GRID,),
    in_specs=[_degp_spec, _p_spec, _row_spec, _b_spec],
    out_specs=_row_spec,
    out_shape=_out_struct,
)


def kernel(x, edge_index, W1, b1, W2, b2):
    src = edge_index[0].reshape(NW, WPW, B)
    dst = edge_index[1].reshape(NW, NCHUNK, CH, B)
    degp = _deg_call(dst)
    g1 = _mm1(degp, x, W1)
    p1 = _agg_call(g1, src, dst)
    g2 = _l2(degp, p1, g1, b1.reshape(1, D), W2)
    p2 = _agg_call(g2, src, dst)
    return _fin(degp, p2, g2, b2.reshape(1, D))


# speedup vs baseline: 26.3668x; 1.0020x over previous
import functools

import jax
import jax.numpy as jnp
from jax import lax
from jax.experimental import pallas as pl
from jax.experimental.pallas import tpu as pltpu
from jax.experimental.pallas import tpu_sc as plsc

N = 10000
E = 320000
D = 128

NC = 2
NS = 16
NW = NC * NS
EPW = E // NW
B = 125
WPW = EPW // B
CH = 8
NCHUNK = WPW // CH
NPAD = 10240
RPT = NPAD // NS
BR = 400
GRID = N // BR

_mesh = plsc.VectorSubcoreMesh(core_axis_name="c", subcore_axis_name="s")


DW = 16
IQ = 8


def _deg_body(dstw_hbm, out_hbm, idx_v, ones_v, zbuf_v, deg_sh):
    c = lax.axis_index("c")
    s = lax.axis_index("s")
    w = c * NS + s
    pltpu.sync_copy(dstw_hbm.at[w], idx_v)

    def _ones(k, carry):
        ones_v[k, :] = jnp.ones((DW,), jnp.float32)
        return carry
    lax.fori_loop(0, B, _ones, 0)

    def _z(i, carry):
        zbuf_v[i, :] = jnp.zeros((DW,), jnp.float32)
        return carry
    lax.fori_loop(0, 40, _z, 0)

    r0 = s * RPT

    def _zc(i, carry):
        pltpu.sync_copy(zbuf_v, deg_sh.at[pl.ds(r0 + i * 40, 40)])
        return carry
    lax.fori_loop(0, RPT // 40, _zc, 0)

    plsc.subcore_barrier()

    def _win(ch, carry):
        def scoped(*sems):
            for k in range(CH):
                pltpu.async_copy(ones_v, deg_sh.at[idx_v.at[ch, k]],
                                 sems[k], add=True)
            for k in range(CH):
                pltpu.make_async_copy(ones_v, deg_sh.at[idx_v.at[ch, k]],
                                      sems[k]).wait()
        pl.run_scoped(scoped, *([pltpu.SemaphoreType.DMA] * CH))
        return carry
    lax.fori_loop(0, NCHUNK, _win, 0)

    plsc.subcore_barrier()

    @pl.when(s < NS - 1)
    def _dump_full():
        pltpu.sync_copy(deg_sh.at[pl.ds(r0, RPT)],
                        out_hbm.at[c, pl.ds(r0, RPT)])

    @pl.when(s == NS - 1)
    def _dump_tail():
        pltpu.sync_copy(deg_sh.at[pl.ds(r0, N - (NS - 1) * RPT)],
                        out_hbm.at[c, pl.ds(r0, N - (NS - 1) * RPT)])


_deg_call = functools.partial(
    pl.kernel,
    out_type=jax.ShapeDtypeStruct((NC, N, DW), jnp.float32),
    mesh=_mesh,
    scratch_types=[
        pltpu.VMEM((NCHUNK, CH, B), jnp.int32),
        pltpu.VMEM((B, DW), jnp.float32),
        pltpu.VMEM((40, DW), jnp.float32),
        pltpu.VMEM_SHARED((NPAD, DW), jnp.float32),
    ],
)(_deg_body)


def _agg_body(g_hbm, srcw_hbm, dstw_hbm, out_hbm,
              srci_v, dstc_v, rows_v, acc_sh):
    c = lax.axis_index("c")
    s = lax.axis_index("s")
    w = c * NS + s
    pltpu.sync_copy(srcw_hbm.at[w], srci_v)

    def _z(i, carry):
        for jj in range(D // 16):
            rows_v[0, i, pl.ds(jj * 16, 16)] = jnp.zeros((16,), jnp.float32)
        return carry
    lax.fori_loop(0, 64, _z, 0)

    r0 = s * RPT

    def _zc(i, carry):
        pltpu.sync_copy(rows_v.at[0, pl.ds(0, 64)],
                        acc_sh.at[pl.ds(r0 + i * 64, 64)])
        return carry
    lax.fori_loop(0, RPT // 64, _zc, 0)

    plsc.subcore_barrier()

    def _start(j, k):
        pltpu.async_copy(g_hbm.at[srci_v.at[j]], rows_v.at[k], gsems[k])

    def _gwait(k):
        pltpu.make_async_copy(g_hbm.at[srci_v.at[0]], rows_v.at[k],
                              gsems[k]).wait()

    def _scat(i, k):
        pltpu.sync_copy(rows_v.at[k], acc_sh.at[dstc_v.at[i]], add=True)

    def _overlap(jg, guard, kg, i_scat, k_scat):
        def scoped(sem):
            @pl.when(guard)
            def _():
                pltpu.async_copy(g_hbm.at[srci_v.at[jg]],
                                 rows_v.at[kg], sem)
            _scat(i_scat, k_scat)

            @pl.when(guard)
            def _():
                pltpu.make_async_copy(g_hbm.at[srci_v.at[jg]],
                                      rows_v.at[kg], sem).wait()
        pl.run_scoped(scoped, pltpu.SemaphoreType.DMA)

    pltpu.sync_copy(g_hbm.at[srci_v.at[0]], rows_v.at[0])

    def _chunk(ch, carry):
        pltpu.sync_copy(dstw_hbm.at[w, ch], dstc_v)

        def _pair(i, carry2):
            j0 = ch * CH + 2 * i
            _overlap(j0 + 1, j0 + 1 < WPW, 1, 2 * i, 0)
            _overlap(j0 + 2, j0 + 2 < WPW, 0, 2 * i + 1, 1)
            return carry2
        lax.fori_loop(0, CH // 2, _pair, 0)
        return carry
    lax.fori_loop(0, NCHUNK, _chunk, 0)

    plsc.subcore_barrier()

    @pl.when(s < NS - 1)
    def _dump_full():
        pltpu.sync_copy(acc_sh.at[pl.ds(r0, RPT)],
                        out_hbm.at[c, pl.ds(r0, RPT)])

    @pl.when(s == NS - 1)
    def _dump_tail():
        pltpu.sync_copy(acc_sh.at[pl.ds(r0, N - (NS - 1) * RPT)],
                        out_hbm.at[c, pl.ds(r0, N - (NS - 1) * RPT)])


_agg_call = functools.partial(
    pl.kernel,
    out_type=jax.ShapeDtypeStruct((NC, N, D), jnp.float32),
    mesh=_mesh,
    scratch_types=[
        pltpu.VMEM((WPW, B), jnp.int32),
        pltpu.VMEM((CH, B), jnp.int32),
        pltpu.VMEM((2, B, D), jnp.float32),
        pltpu.VMEM_SHARED((NPAD, D), jnp.float32),
    ],
)(_agg_body)


def _dinv_block(degp_ref):
    deg = 1.0 + degp_ref[0, :, 0:1] + degp_ref[1, :, 0:1]
    return lax.rsqrt(deg)


def _mm1_body(degp_ref, x_ref, w_ref, o_ref):
    dinv = _dinv_block(degp_ref)
    h = jnp.dot(x_ref[...], w_ref[...], preferred_element_type=jnp.float32)
    o_ref[...] = h * dinv


def _l2_body(degp_ref, p_ref, g_ref, b_ref, w_ref, o_ref):
    dinv = _dinv_block(degp_ref)
    ssum = p_ref[0] + p_ref[1] + g_ref[...]
    h1 = jnp.maximum(ssum * dinv + b_ref[...], 0.0)
    o_ref[...] = jnp.dot(h1, w_ref[...],
                         preferred_element_type=jnp.float32) * dinv


def _fin_body(degp_ref, p_ref, g_ref, b_ref, o_ref):
    dinv = _dinv_block(degp_ref)
    ssum = p_ref[0] + p_ref[1] + g_ref[...]
    o_ref[...] = ssum * dinv + b_ref[...]


_degp_spec = pl.BlockSpec((NC, BR, DW), lambda i: (0, i, 0))
_row_spec = pl.BlockSpec((BR, D), lambda i: (i, 0))
_p_spec = pl.BlockSpec((NC, BR, D), lambda i: (0, i, 0))
_w_spec = pl.BlockSpec((D, D), lambda i: (0, 0))
_b_spec = pl.BlockSpec((1, D), lambda i: (0, 0))
_out_struct = jax.ShapeDtypeStruct((N, D), jnp.float32)

_mm1 = pl.pallas_call(
    _mm1_body,
    grid=(GRID,),
    in_specs=[_degp_spec, _row_spec, _w_spec],
    out_specs=_row_spec,
    out_shape=_out_struct,
)

_l2 = pl.pallas_call(
    _l2_body,
    grid=(GRID,),
    in_specs=[_degp_spec, _p_spec, _row_spec, _b_spec, _w_spec],
    out_specs=_row_spec,
    out_shape=_out_struct,
)

_fin = pl.pallas_call(
    _fin_body,
    grid=(GRID,),
    in_specs=[_degp_spec, _p_spec, _row_spec, _b_spec],
    out_specs=_row_spec,
    out_shape=_out_struct,
)


def kernel(x, edge_index, W1, b1, W2, b2):
    src = edge_index[0].reshape(NW, WPW, B)
    dst = edge_index[1].reshape(NW, NCHUNK, CH, B)
    degp = _deg_call(dst)
    g1 = _mm1(degp, x, W1)
    p1 = _agg_call(g1, src, dst)
    g2 = _l2(degp, p1, g1, b1.reshape(1, D), W2)
    p2 = _agg_call(g2, src, dst)
    return _fin(degp, p2, g2, b2.reshape(1, D))
